# Initial kernel scaffold; baseline (speedup 1.0000x reference)
import jax
import jax.numpy as jnp
from jax.experimental import pallas as pl


def kernel(x, edge_index, batch, l0_W1, l0_b1, l0_W2, l0_b2, l1_W1, l1_b1, l1_W2, l1_b2, l2_W1, l2_b1, l2_W2, l2_b2, proj_W, proj_b):
    raise NotImplementedError("write your pallas kernel here")


# speedup vs baseline: 6.0517x; 6.0517x over previous
import functools

import jax
import jax.numpy as jnp
from jax import lax
from jax.experimental import pallas as pl
from jax.experimental.pallas import tpu as pltpu
from jax.experimental.pallas import tpu_sc as plsc

_G = 512
_GPAD = 520
_BLK = 512


def _mesh():
    return plsc.VectorSubcoreMesh(core_axis_name="c", subcore_axis_name="s",
                                  num_cores=2, num_subcores=16)


def _sc_agg8(x_pad, src3d, dst3d, zrows):
    npad = x_pad.shape[0]
    ngroups = src3d.shape[0]
    group = src3d.shape[1]
    groups = ngroups // 32
    rpt = npad // 16

    @functools.partial(
        pl.kernel,
        out_type=jax.ShapeDtypeStruct((2, npad, 8), jnp.float32),
        mesh=_mesh(),
        compiler_params=pltpu.CompilerParams(use_tc_tiling_on_sc=False),
        scratch_types=[
            pltpu.VMEM((group, 128), jnp.int32),
            pltpu.VMEM((group, 128), jnp.int32),
            pltpu.VMEM((group * 128, 8), jnp.float32),
            pltpu.VMEM_SHARED((npad, 8), jnp.float32),
            pltpu.SemaphoreType.DMA,
        ],
    )
    def k(x_hbm, src_hbm, dst_hbm, z_hbm, out_hbm, sidx, didx, rows, acc, gsem):
        c = lax.axis_index("c")
        s = lax.axis_index("s")
        pltpu.sync_copy(z_hbm, acc.at[pl.ds(s * rpt, rpt)])
        plsc.subcore_barrier()

        w = c * 16 + s

        @pl.loop(0, groups)
        def _(g):
            gi = w * groups + g
            pltpu.sync_copy(src_hbm.at[gi], sidx)
            pltpu.sync_copy(dst_hbm.at[gi], didx)
            descs = [
                pltpu.async_copy(x_hbm.at[sidx.at[j]],
                                 rows.at[pl.ds(j * 128, 128)], gsem)
                for j in range(group)
            ]
            for d in descs:
                d.wait()
            for j in range(group):
                pltpu.sync_copy(rows.at[pl.ds(j * 128, 128)],
                                acc.at[didx.at[j]], add=True)

        plsc.subcore_barrier()

        @pl.when(c == 0)
        def _():
            pltpu.sync_copy(acc.at[pl.ds(s * rpt, rpt)],
                            out_hbm.at[0].at[pl.ds(s * rpt, rpt)])

        @pl.when(c == 1)
        def _():
            pltpu.sync_copy(acc.at[pl.ds(s * rpt, rpt)],
                            out_hbm.at[1].at[pl.ds(s * rpt, rpt)])

    return k(x_pad, src3d, dst3d, zrows)


def _sc_agg64(h_split, src3d, dst3d, zrows):
    npad = h_split.shape[1]
    ngroups = src3d.shape[0]
    group = src3d.shape[1]
    groups = ngroups // 16
    rpt = npad // 16

    @functools.partial(
        pl.kernel,
        out_type=jax.ShapeDtypeStruct((2, npad, 32), jnp.float32),
        mesh=_mesh(),
        compiler_params=pltpu.CompilerParams(use_tc_tiling_on_sc=False),
        scratch_types=[
            pltpu.VMEM((group, 128), jnp.int32),
            pltpu.VMEM((group, 128), jnp.int32),
            pltpu.VMEM((group // 2 * 128, 32), jnp.float32),
            pltpu.VMEM_SHARED((npad, 32), jnp.float32),
            pltpu.SemaphoreType.DMA,
        ],
    )
    def k(h_hbm, src_hbm, dst_hbm, z_hbm, out_hbm, sidx, didx, rows, acc, gsem):
        c = lax.axis_index("c")
        s = lax.axis_index("s")
        pltpu.sync_copy(z_hbm, acc.at[pl.ds(s * rpt, rpt)])
        plsc.subcore_barrier()

        half = group // 2

        def run(h_ref):
            @pl.loop(0, groups)
            def _(g):
                gi = s * groups + g
                pltpu.sync_copy(src_hbm.at[gi], sidx)
                pltpu.sync_copy(dst_hbm.at[gi], didx)
                for r in range(2):
                    descs = [
                        pltpu.async_copy(h_ref.at[sidx.at[r * half + j]],
                                         rows.at[pl.ds(j * 128, 128)], gsem)
                        for j in range(half)
                    ]
                    for d in descs:
                        d.wait()
                    for j in range(half):
                        pltpu.sync_copy(rows.at[pl.ds(j * 128, 128)],
                                        acc.at[didx.at[r * half + j]], add=True)

        @pl.when(c == 0)
        def _():
            run(h_hbm.at[0])

        @pl.when(c == 1)
        def _():
            run(h_hbm.at[1])

        plsc.subcore_barrier()

        @pl.when(c == 0)
        def _():
            pltpu.sync_copy(acc.at[pl.ds(s * rpt, rpt)],
                            out_hbm.at[0].at[pl.ds(s * rpt, rpt)])

        @pl.when(c == 1)
        def _():
            pltpu.sync_copy(acc.at[pl.ds(s * rpt, rpt)],
                            out_hbm.at[1].at[pl.ds(s * rpt, rpt)])

    return k(h_split, src3d, dst3d, zrows)


def _sc_pool(h3, batch3d, zpool):
    npad = h3.shape[0]
    per_w = npad // 32
    per_w_chunks = per_w // 128

    @functools.partial(
        pl.kernel,
        out_type=jax.ShapeDtypeStruct((2, _GPAD, 64), jnp.float32),
        mesh=_mesh(),
        compiler_params=pltpu.CompilerParams(use_tc_tiling_on_sc=False),
        scratch_types=[
            pltpu.VMEM((per_w_chunks, 128), jnp.int32),
            pltpu.VMEM((per_w, 64), jnp.float32),
            pltpu.VMEM_SHARED((_GPAD, 64), jnp.float32),
        ],
    )
    def k(h_hbm, b_hbm, z_hbm, out_hbm, bidx, hrows, acc):
        c = lax.axis_index("c")
        s = lax.axis_index("s")

        @pl.when(s == 0)
        def _():
            pltpu.sync_copy(z_hbm, acc)

        plsc.subcore_barrier()

        w = c * 16 + s
        pltpu.sync_copy(b_hbm.at[w], bidx)
        pltpu.sync_copy(h_hbm.at[pl.ds(w * per_w, per_w)], hrows)
        for j in range(per_w_chunks):
            pltpu.sync_copy(hrows.at[pl.ds(j * 128, 128)],
                            acc.at[bidx.at[j]], add=True)

        plsc.subcore_barrier()

        @pl.when(jnp.logical_and(c == 0, s == 0))
        def _():
            pltpu.sync_copy(acc, out_hbm.at[0])

        @pl.when(jnp.logical_and(c == 1, s == 0))
        def _():
            pltpu.sync_copy(acc, out_hbm.at[1])

    return k(h3, batch3d, zpool)


def _tc_mlp0(x_pad, a0, w1, b1, w2, b2, n_valid):
    npad = x_pad.shape[0]

    def body(x_ref, a_ref, w1_ref, b1_ref, w2_ref, b2_ref, o_ref):
        i = pl.program_id(0)
        h = x_ref[...] + a_ref[0] + a_ref[1]
        z = jnp.dot(h, w1_ref[...], preferred_element_type=jnp.float32)
        z = jnp.maximum(z + b1_ref[...], 0.0)
        z = jnp.dot(z, w2_ref[...], preferred_element_type=jnp.float32)
        z = jnp.maximum(z + b2_ref[...], 0.0)
        rows = i * _BLK + lax.broadcasted_iota(jnp.int32, (_BLK, 1), 0)
        z = jnp.where(rows < n_valid, z, 0.0)
        o_ref[0] = z[:, :32]
        o_ref[1] = z[:, 32:]

    return pl.pallas_call(
        body,
        grid=(npad // _BLK,),
        in_specs=[
            pl.BlockSpec((_BLK, 8), lambda i: (i, 0)),
            pl.BlockSpec((2, _BLK, 8), lambda i: (0, i, 0)),
            pl.BlockSpec((8, 64), lambda i: (0, 0)),
            pl.BlockSpec((1, 64), lambda i: (0, 0)),
            pl.BlockSpec((64, 64), lambda i: (0, 0)),
            pl.BlockSpec((1, 64), lambda i: (0, 0)),
        ],
        out_specs=pl.BlockSpec((2, _BLK, 32), lambda i: (0, i, 0)),
        out_shape=jax.ShapeDtypeStruct((2, npad, 32), jnp.float32),
    )(x_pad, a0, w1, b1, w2, b2)


def _tc_mlp(h_split, a_split, w1, b1, w2, b2, n_valid, split_out):
    npad = h_split.shape[1]

    def body(h_ref, a_ref, w1_ref, b1_ref, w2_ref, b2_ref, o_ref):
        i = pl.program_id(0)
        h = jnp.concatenate([h_ref[0] + a_ref[0], h_ref[1] + a_ref[1]], axis=1)
        z = jnp.dot(h, w1_ref[...], preferred_element_type=jnp.float32)
        z = jnp.maximum(z + b1_ref[...], 0.0)
        z = jnp.dot(z, w2_ref[...], preferred_element_type=jnp.float32)
        z = jnp.maximum(z + b2_ref[...], 0.0)
        rows = i * _BLK + lax.broadcasted_iota(jnp.int32, (_BLK, 1), 0)
        z = jnp.where(rows < n_valid, z, 0.0)
        if split_out:
            o_ref[0] = z[:, :32]
            o_ref[1] = z[:, 32:]
        else:
            o_ref[...] = z

    if split_out:
        out_spec = pl.BlockSpec((2, _BLK, 32), lambda i: (0, i, 0))
        out_shape = jax.ShapeDtypeStruct((2, npad, 32), jnp.float32)
    else:
        out_spec = pl.BlockSpec((_BLK, 64), lambda i: (i, 0))
        out_shape = jax.ShapeDtypeStruct((npad, 64), jnp.float32)

    return pl.pallas_call(
        body,
        grid=(npad // _BLK,),
        in_specs=[
            pl.BlockSpec((2, _BLK, 32), lambda i: (0, i, 0)),
            pl.BlockSpec((2, _BLK, 32), lambda i: (0, i, 0)),
            pl.BlockSpec((64, 64), lambda i: (0, 0)),
            pl.BlockSpec((1, 64), lambda i: (0, 0)),
            pl.BlockSpec((64, 64), lambda i: (0, 0)),
            pl.BlockSpec((1, 64), lambda i: (0, 0)),
        ],
        out_specs=out_spec,
        out_shape=out_shape,
    )(h_split, a_split, w1, b1, w2, b2)


def _tc_proj(pooled, w, b):
    def body(p_ref, w_ref, b_ref, o_ref):
        p = p_ref[0, :_G] + p_ref[1, :_G]
        o_ref[...] = jnp.dot(p, w_ref[...],
                             preferred_element_type=jnp.float32) + b_ref[...]

    return pl.pallas_call(
        body,
        out_shape=jax.ShapeDtypeStruct((_G, pooled.shape[2]), jnp.float32),
    )(pooled, w, b)


def kernel(x, edge_index, batch,
           l0_W1, l0_b1, l0_W2, l0_b2,
           l1_W1, l1_b1, l1_W2, l1_b2,
           l2_W1, l2_b1, l2_W2, l2_b2,
           proj_W, proj_b):
    n, in_dim = x.shape
    e = edge_index.shape[1]
    f32 = jnp.float32

    npad = ((n + 4096) // 4096) * 4096
    epad = ((e + 32767) // 32768) * 32768

    x_pad = jnp.zeros((npad, 8), f32).at[:n, :in_dim].set(x)
    fill = jnp.full((epad - e,), n, jnp.int32)
    src3d = jnp.concatenate([edge_index[0], fill]).reshape(-1, 8, 128)
    dst3d = jnp.concatenate([edge_index[1], fill]).reshape(-1, 8, 128)
    batch3d = jnp.concatenate(
        [batch.astype(jnp.int32), jnp.full((npad - n,), _G, jnp.int32)]
    ).reshape(32, -1, 128)
    z8 = jnp.zeros((npad // 16, 8), f32)
    z32 = jnp.zeros((npad // 16, 32), f32)
    zp = jnp.zeros((_GPAD, 64), f32)
    w1p = jnp.zeros((8, 64), f32).at[:in_dim].set(l0_W1)
    b = lambda v: v.reshape(1, -1)

    a0 = _sc_agg8(x_pad, src3d, dst3d, z8)
    h1 = _tc_mlp0(x_pad, a0, w1p, b(l0_b1), l0_W2, b(l0_b2), n)
    a1 = _sc_agg64(h1, src3d, dst3d, z32)
    h2 = _tc_mlp(h1, a1, l1_W1, b(l1_b1), l1_W2, b(l1_b2), n, split_out=True)
    a2 = _sc_agg64(h2, src3d, dst3d, z32)
    h3 = _tc_mlp(h2, a2, l2_W1, b(l2_b1), l2_W2, b(l2_b2), n, split_out=False)
    p = _sc_pool(h3, batch3d, zp)
    return _tc_proj(p, proj_W, b(proj_b))


# speedup vs baseline: 6.6662x; 1.1015x over previous
import functools

import jax
import jax.numpy as jnp
from jax import lax
from jax.experimental import pallas as pl
from jax.experimental.pallas import tpu as pltpu
from jax.experimental.pallas import tpu_sc as plsc

_G = 512
_GPAD = 520
_BLK = 512


def _mesh():
    return plsc.VectorSubcoreMesh(core_axis_name="c", subcore_axis_name="s",
                                  num_cores=2, num_subcores=16)


def _edge_pipeline(h_ref, src_hbm, dst_hbm, acc, rows,
                   sA, dA, sB, dB, isem, gsem, s0, s1,
                   base, pairs_k, steps, cps):
    def wait_scatter(did, p):
        sem = s0 if p == 0 else s1
        for _ in range(cps):
            pltpu.make_async_copy(rows.at[pl.ds(0, 128)],
                                  acc.at[did.at[0]], sem).wait()

    def step(sid, did, st, cond):
        p = st % 2
        slot0 = p * cps
        if cond is None:
            wait_scatter(did, p)
        else:
            @pl.when(cond)
            def _():
                wait_scatter(did, p)
        descs = [
            pltpu.async_copy(h_ref.at[sid.at[st * cps + i]],
                             rows.at[pl.ds((slot0 + i) * 128, 128)], gsem)
            for i in range(cps)
        ]
        for dsc in descs:
            dsc.wait()
        sem = s0 if p == 0 else s1
        for i in range(cps):
            pltpu.async_copy(rows.at[pl.ds((slot0 + i) * 128, 128)],
                             acc.at[did.at[st * cps + i]], sem, add=True)

    def wait_idx(buf_s, buf_d):
        pltpu.make_async_copy(src_hbm.at[0], buf_s, isem).wait()
        pltpu.make_async_copy(src_hbm.at[0], buf_d, isem).wait()

    pltpu.sync_copy(src_hbm.at[base], sA)
    pltpu.sync_copy(dst_hbm.at[base], dA)

    @pl.loop(0, pairs_k)
    def _(kk):
        gB = base + 2 * kk + 1
        pltpu.async_copy(src_hbm.at[gB], sB, isem)
        pltpu.async_copy(dst_hbm.at[gB], dB, isem)
        for st in range(steps):
            step(sA, dA, st, (kk > 0) if st < 2 else None)
        wait_idx(sB, dB)

        @pl.when(kk < pairs_k - 1)
        def _():
            gA2 = base + 2 * kk + 2
            pltpu.async_copy(src_hbm.at[gA2], sA, isem)
            pltpu.async_copy(dst_hbm.at[gA2], dA, isem)

        for st in range(steps):
            step(sB, dB, st, None)

        @pl.when(kk < pairs_k - 1)
        def _():
            wait_idx(sA, dA)

    wait_scatter(dA, 0)
    wait_scatter(dA, 1)


def _sc_agg8(x_pad, src3d, dst3d, zrows):
    npad = x_pad.shape[0]
    ngroups = src3d.shape[0]
    gpc = src3d.shape[1]
    groups = ngroups // 32
    cps = gpc // 2
    steps = 2
    rpt = npad // 16

    @functools.partial(
        pl.kernel,
        out_type=jax.ShapeDtypeStruct((2, npad, 8), jnp.float32),
        mesh=_mesh(),
        compiler_params=pltpu.CompilerParams(use_tc_tiling_on_sc=False),
        scratch_types=[
            pltpu.VMEM((gpc, 128), jnp.int32),
            pltpu.VMEM((gpc, 128), jnp.int32),
            pltpu.VMEM((gpc, 128), jnp.int32),
            pltpu.VMEM((gpc, 128), jnp.int32),
            pltpu.VMEM((2 * cps * 128, 8), jnp.float32),
            pltpu.VMEM_SHARED((npad, 8), jnp.float32),
            pltpu.SemaphoreType.DMA,
            pltpu.SemaphoreType.DMA,
            pltpu.SemaphoreType.DMA,
            pltpu.SemaphoreType.DMA,
        ],
    )
    def k(x_hbm, src_hbm, dst_hbm, z_hbm, out_hbm,
          sA, dA, sB, dB, rows, acc, isem, gsem, s0, s1):
        c = lax.axis_index("c")
        s = lax.axis_index("s")
        pltpu.sync_copy(z_hbm, acc.at[pl.ds(s * rpt, rpt)])
        plsc.subcore_barrier()

        w = c * 16 + s
        _edge_pipeline(x_hbm, src_hbm, dst_hbm, acc, rows,
                       sA, dA, sB, dB, isem, gsem, s0, s1,
                       base=w * groups, pairs_k=groups // 2,
                       steps=steps, cps=cps)

        plsc.subcore_barrier()

        @pl.when(c == 0)
        def _():
            pltpu.sync_copy(acc.at[pl.ds(s * rpt, rpt)],
                            out_hbm.at[0].at[pl.ds(s * rpt, rpt)])

        @pl.when(c == 1)
        def _():
            pltpu.sync_copy(acc.at[pl.ds(s * rpt, rpt)],
                            out_hbm.at[1].at[pl.ds(s * rpt, rpt)])

    return k(x_pad, src3d, dst3d, zrows)


def _sc_agg64(h_split, src3d, dst3d, zrows):
    npad = h_split.shape[1]
    ngroups = src3d.shape[0]
    gpc = src3d.shape[1]
    groups = ngroups // 16
    cps = 2
    steps = gpc // cps
    rpt = npad // 16

    @functools.partial(
        pl.kernel,
        out_type=jax.ShapeDtypeStruct((2, npad, 32), jnp.float32),
        mesh=_mesh(),
        compiler_params=pltpu.CompilerParams(use_tc_tiling_on_sc=False),
        scratch_types=[
            pltpu.VMEM((gpc, 128), jnp.int32),
            pltpu.VMEM((gpc, 128), jnp.int32),
            pltpu.VMEM((gpc, 128), jnp.int32),
            pltpu.VMEM((gpc, 128), jnp.int32),
            pltpu.VMEM((2 * cps * 128, 32), jnp.float32),
            pltpu.VMEM_SHARED((npad, 32), jnp.float32),
            pltpu.SemaphoreType.DMA,
            pltpu.SemaphoreType.DMA,
            pltpu.SemaphoreType.DMA,
            pltpu.SemaphoreType.DMA,
        ],
    )
    def k(h_hbm, src_hbm, dst_hbm, z_hbm, out_hbm,
          sA, dA, sB, dB, rows, acc, isem, gsem, s0, s1):
        c = lax.axis_index("c")
        s = lax.axis_index("s")
        pltpu.sync_copy(z_hbm, acc.at[pl.ds(s * rpt, rpt)])
        plsc.subcore_barrier()

        def run(h_ref):
            _edge_pipeline(h_ref, src_hbm, dst_hbm, acc, rows,
                           sA, dA, sB, dB, isem, gsem, s0, s1,
                           base=s * groups, pairs_k=groups // 2,
                           steps=steps, cps=cps)

        @pl.when(c == 0)
        def _():
            run(h_hbm.at[0])

        @pl.when(c == 1)
        def _():
            run(h_hbm.at[1])

        plsc.subcore_barrier()

        @pl.when(c == 0)
        def _():
            pltpu.sync_copy(acc.at[pl.ds(s * rpt, rpt)],
                            out_hbm.at[0].at[pl.ds(s * rpt, rpt)])

        @pl.when(c == 1)
        def _():
            pltpu.sync_copy(acc.at[pl.ds(s * rpt, rpt)],
                            out_hbm.at[1].at[pl.ds(s * rpt, rpt)])

    return k(h_split, src3d, dst3d, zrows)


def _sc_pool(h3, batch3d, zpool):
    npad = h3.shape[0]
    per_w = npad // 32
    per_w_chunks = per_w // 128

    @functools.partial(
        pl.kernel,
        out_type=jax.ShapeDtypeStruct((2, _GPAD, 64), jnp.float32),
        mesh=_mesh(),
        compiler_params=pltpu.CompilerParams(use_tc_tiling_on_sc=False),
        scratch_types=[
            pltpu.VMEM((per_w_chunks, 128), jnp.int32),
            pltpu.VMEM((per_w, 64), jnp.float32),
            pltpu.VMEM_SHARED((_GPAD, 64), jnp.float32),
        ],
    )
    def k(h_hbm, b_hbm, z_hbm, out_hbm, bidx, hrows, acc):
        c = lax.axis_index("c")
        s = lax.axis_index("s")

        @pl.when(s == 0)
        def _():
            pltpu.sync_copy(z_hbm, acc)

        plsc.subcore_barrier()

        w = c * 16 + s
        pltpu.sync_copy(b_hbm.at[w], bidx)
        pltpu.sync_copy(h_hbm.at[pl.ds(w * per_w, per_w)], hrows)
        for j in range(per_w_chunks):
            pltpu.sync_copy(hrows.at[pl.ds(j * 128, 128)],
                            acc.at[bidx.at[j]], add=True)

        plsc.subcore_barrier()

        @pl.when(jnp.logical_and(c == 0, s == 0))
        def _():
            pltpu.sync_copy(acc, out_hbm.at[0])

        @pl.when(jnp.logical_and(c == 1, s == 0))
        def _():
            pltpu.sync_copy(acc, out_hbm.at[1])

    return k(h3, batch3d, zpool)


def _tc_mlp0(x_pad, a0, w1, b1, w2, b2, n_valid):
    npad = x_pad.shape[0]

    def body(x_ref, a_ref, w1_ref, b1_ref, w2_ref, b2_ref, o_ref):
        i = pl.program_id(0)
        h = x_ref[...] + a_ref[0] + a_ref[1]
        z = jnp.dot(h, w1_ref[...], preferred_element_type=jnp.float32)
        z = jnp.maximum(z + b1_ref[...], 0.0)
        z = jnp.dot(z, w2_ref[...], preferred_element_type=jnp.float32)
        z = jnp.maximum(z + b2_ref[...], 0.0)
        rows = i * _BLK + lax.broadcasted_iota(jnp.int32, (_BLK, 1), 0)
        z = jnp.where(rows < n_valid, z, 0.0)
        o_ref[0] = z[:, :32]
        o_ref[1] = z[:, 32:]

    return pl.pallas_call(
        body,
        grid=(npad // _BLK,),
        in_specs=[
            pl.BlockSpec((_BLK, 8), lambda i: (i, 0)),
            pl.BlockSpec((2, _BLK, 8), lambda i: (0, i, 0)),
            pl.BlockSpec((8, 64), lambda i: (0, 0)),
            pl.BlockSpec((1, 64), lambda i: (0, 0)),
            pl.BlockSpec((64, 64), lambda i: (0, 0)),
            pl.BlockSpec((1, 64), lambda i: (0, 0)),
        ],
        out_specs=pl.BlockSpec((2, _BLK, 32), lambda i: (0, i, 0)),
        out_shape=jax.ShapeDtypeStruct((2, npad, 32), jnp.float32),
    )(x_pad, a0, w1, b1, w2, b2)


def _tc_mlp(h_split, a_split, w1, b1, w2, b2, n_valid, split_out):
    npad = h_split.shape[1]

    def body(h_ref, a_ref, w1_ref, b1_ref, w2_ref, b2_ref, o_ref):
        i = pl.program_id(0)
        h = jnp.concatenate([h_ref[0] + a_ref[0], h_ref[1] + a_ref[1]], axis=1)
        z = jnp.dot(h, w1_ref[...], preferred_element_type=jnp.float32)
        z = jnp.maximum(z + b1_ref[...], 0.0)
        z = jnp.dot(z, w2_ref[...], preferred_element_type=jnp.float32)
        z = jnp.maximum(z + b2_ref[...], 0.0)
        rows = i * _BLK + lax.broadcasted_iota(jnp.int32, (_BLK, 1), 0)
        z = jnp.where(rows < n_valid, z, 0.0)
        if split_out:
            o_ref[0] = z[:, :32]
            o_ref[1] = z[:, 32:]
        else:
            o_ref[...] = z

    if split_out:
        out_spec = pl.BlockSpec((2, _BLK, 32), lambda i: (0, i, 0))
        out_shape = jax.ShapeDtypeStruct((2, npad, 32), jnp.float32)
    else:
        out_spec = pl.BlockSpec((_BLK, 64), lambda i: (i, 0))
        out_shape = jax.ShapeDtypeStruct((npad, 64), jnp.float32)

    return pl.pallas_call(
        body,
        grid=(npad // _BLK,),
        in_specs=[
            pl.BlockSpec((2, _BLK, 32), lambda i: (0, i, 0)),
            pl.BlockSpec((2, _BLK, 32), lambda i: (0, i, 0)),
            pl.BlockSpec((64, 64), lambda i: (0, 0)),
            pl.BlockSpec((1, 64), lambda i: (0, 0)),
            pl.BlockSpec((64, 64), lambda i: (0, 0)),
            pl.BlockSpec((1, 64), lambda i: (0, 0)),
        ],
        out_specs=out_spec,
        out_shape=out_shape,
    )(h_split, a_split, w1, b1, w2, b2)


def _tc_proj(pooled, w, b):
    def body(p_ref, w_ref, b_ref, o_ref):
        p = p_ref[0, :_G] + p_ref[1, :_G]
        o_ref[...] = jnp.dot(p, w_ref[...],
                             preferred_element_type=jnp.float32) + b_ref[...]

    return pl.pallas_call(
        body,
        out_shape=jax.ShapeDtypeStruct((_G, pooled.shape[2]), jnp.float32),
    )(pooled, w, b)


def kernel(x, edge_index, batch,
           l0_W1, l0_b1, l0_W2, l0_b2,
           l1_W1, l1_b1, l1_W2, l1_b2,
           l2_W1, l2_b1, l2_W2, l2_b2,
           proj_W, proj_b):
    n, in_dim = x.shape
    e = edge_index.shape[1]
    f32 = jnp.float32

    npad = ((n + 4096) // 4096) * 4096
    epad = ((e + 32767) // 32768) * 32768

    x_pad = jnp.zeros((npad, 8), f32).at[:n, :in_dim].set(x)
    fill = jnp.full((epad - e,), n, jnp.int32)
    src = jnp.concatenate([edge_index[0], fill])
    dst = jnp.concatenate([edge_index[1], fill])
    src4, dst4 = src.reshape(-1, 4, 128), dst.reshape(-1, 4, 128)
    src8, dst8 = src.reshape(-1, 8, 128), dst.reshape(-1, 8, 128)
    batch3d = jnp.concatenate(
        [batch.astype(jnp.int32), jnp.full((npad - n,), _G, jnp.int32)]
    ).reshape(32, -1, 128)
    z8 = jnp.zeros((npad // 16, 8), f32)
    z32 = jnp.zeros((npad // 16, 32), f32)
    zp = jnp.zeros((_GPAD, 64), f32)
    w1p = jnp.zeros((8, 64), f32).at[:in_dim].set(l0_W1)
    b = lambda v: v.reshape(1, -1)

    a0 = _sc_agg8(x_pad, src4, dst4, z8)
    h1 = _tc_mlp0(x_pad, a0, w1p, b(l0_b1), l0_W2, b(l0_b2), n)
    a1 = _sc_agg64(h1, src8, dst8, z32)
    h2 = _tc_mlp(h1, a1, l1_W1, b(l1_b1), l1_W2, b(l1_b2), n, split_out=True)
    a2 = _sc_agg64(h2, src8, dst8, z32)
    h3 = _tc_mlp(h2, a2, l2_W1, b(l2_b1), l2_W2, b(l2_b2), n, split_out=False)
    p = _sc_pool(h3, batch3d, zp)
    return _tc_proj(p, proj_W, b(proj_b))


# speedup vs baseline: 8.2140x; 1.2322x over previous
import functools

import jax
import jax.numpy as jnp
from jax import lax
from jax.experimental import pallas as pl
from jax.experimental.pallas import tpu as pltpu
from jax.experimental.pallas import tpu_sc as plsc

_G = 512
_GPAD = 520
_BLK = 512


def _mesh():
    return plsc.VectorSubcoreMesh(core_axis_name="c", subcore_axis_name="s",
                                  num_cores=2, num_subcores=16)


def _edge_pipeline(h_ref, src_hbm, dst_hbm, acc, rows,
                   sA, dA, sB, dB, isem, gsem, s0, s1,
                   base, pairs_k, steps, cps):
    def wait_scatter(did, p):
        sem = s0 if p == 0 else s1
        for _ in range(cps):
            pltpu.make_async_copy(rows.at[pl.ds(0, 128)],
                                  acc.at[did.at[0]], sem).wait()

    def step(sid, did, st, cond):
        p = st % 2
        slot0 = p * cps
        if cond is None:
            wait_scatter(did, p)
        else:
            @pl.when(cond)
            def _():
                wait_scatter(did, p)
        descs = [
            pltpu.async_copy(h_ref.at[sid.at[st * cps + i]],
                             rows.at[pl.ds((slot0 + i) * 128, 128)], gsem)
            for i in range(cps)
        ]
        for dsc in descs:
            dsc.wait()
        sem = s0 if p == 0 else s1
        for i in range(cps):
            pltpu.async_copy(rows.at[pl.ds((slot0 + i) * 128, 128)],
                             acc.at[did.at[st * cps + i]], sem, add=True)

    def wait_idx(buf_s, buf_d):
        pltpu.make_async_copy(src_hbm.at[0], buf_s, isem).wait()
        pltpu.make_async_copy(src_hbm.at[0], buf_d, isem).wait()

    pltpu.sync_copy(src_hbm.at[base], sA)
    pltpu.sync_copy(dst_hbm.at[base], dA)

    @pl.loop(0, pairs_k)
    def _(kk):
        gB = base + 2 * kk + 1
        pltpu.async_copy(src_hbm.at[gB], sB, isem)
        pltpu.async_copy(dst_hbm.at[gB], dB, isem)
        for st in range(steps):
            step(sA, dA, st, (kk > 0) if st < 2 else None)
        wait_idx(sB, dB)

        @pl.when(kk < pairs_k - 1)
        def _():
            gA2 = base + 2 * kk + 2
            pltpu.async_copy(src_hbm.at[gA2], sA, isem)
            pltpu.async_copy(dst_hbm.at[gA2], dA, isem)

        for st in range(steps):
            step(sB, dB, st, None)

        @pl.when(kk < pairs_k - 1)
        def _():
            wait_idx(sA, dA)

    wait_scatter(dA, 0)
    wait_scatter(dA, 1)


def _sc_agg8(x_pad, src3d, dst3d, zrows):
    npad = x_pad.shape[0]
    ngroups = src3d.shape[0]
    gpc = src3d.shape[1]
    groups = ngroups // 32
    cps = gpc // 2
    steps = 2
    rpt = npad // 16

    @functools.partial(
        pl.kernel,
        out_type=jax.ShapeDtypeStruct((2, npad, 8), jnp.float32),
        mesh=_mesh(),
        compiler_params=pltpu.CompilerParams(use_tc_tiling_on_sc=False),
        scratch_types=[
            pltpu.VMEM((gpc, 128), jnp.int32),
            pltpu.VMEM((gpc, 128), jnp.int32),
            pltpu.VMEM((gpc, 128), jnp.int32),
            pltpu.VMEM((gpc, 128), jnp.int32),
            pltpu.VMEM((2 * cps * 128, 8), jnp.float32),
            pltpu.VMEM_SHARED((npad, 8), jnp.float32),
            pltpu.SemaphoreType.DMA,
            pltpu.SemaphoreType.DMA,
            pltpu.SemaphoreType.DMA,
            pltpu.SemaphoreType.DMA,
        ],
    )
    def k(x_hbm, src_hbm, dst_hbm, z_hbm, out_hbm,
          sA, dA, sB, dB, rows, acc, isem, gsem, s0, s1):
        c = lax.axis_index("c")
        s = lax.axis_index("s")
        pltpu.sync_copy(z_hbm, acc.at[pl.ds(s * rpt, rpt)])
        plsc.subcore_barrier()

        w = c * 16 + s
        _edge_pipeline(x_hbm, src_hbm, dst_hbm, acc, rows,
                       sA, dA, sB, dB, isem, gsem, s0, s1,
                       base=w * groups, pairs_k=groups // 2,
                       steps=steps, cps=cps)

        plsc.subcore_barrier()

        @pl.when(c == 0)
        def _():
            pltpu.sync_copy(acc.at[pl.ds(s * rpt, rpt)],
                            out_hbm.at[0].at[pl.ds(s * rpt, rpt)])

        @pl.when(c == 1)
        def _():
            pltpu.sync_copy(acc.at[pl.ds(s * rpt, rpt)],
                            out_hbm.at[1].at[pl.ds(s * rpt, rpt)])

    return k(x_pad, src3d, dst3d, zrows)


def _sc_agg64(h_split, src3d, dst3d, zrows):
    npad = h_split.shape[1]
    ngroups = src3d.shape[0]
    gpc = src3d.shape[1]
    groups = ngroups // 16
    cps = 2
    steps = gpc // cps
    rpt = npad // 16

    @functools.partial(
        pl.kernel,
        out_type=jax.ShapeDtypeStruct((2, npad, 32), jnp.float32),
        mesh=_mesh(),
        compiler_params=pltpu.CompilerParams(use_tc_tiling_on_sc=False),
        scratch_types=[
            pltpu.VMEM((gpc, 128), jnp.int32),
            pltpu.VMEM((gpc, 128), jnp.int32),
            pltpu.VMEM((gpc, 128), jnp.int32),
            pltpu.VMEM((gpc, 128), jnp.int32),
            pltpu.VMEM((2 * cps * 128, 32), jnp.float32),
            pltpu.VMEM_SHARED((npad, 32), jnp.float32),
            pltpu.SemaphoreType.DMA,
            pltpu.SemaphoreType.DMA,
            pltpu.SemaphoreType.DMA,
            pltpu.SemaphoreType.DMA,
        ],
    )
    def k(h_hbm, src_hbm, dst_hbm, z_hbm, out_hbm,
          sA, dA, sB, dB, rows, acc, isem, gsem, s0, s1):
        c = lax.axis_index("c")
        s = lax.axis_index("s")
        pltpu.sync_copy(z_hbm, acc.at[pl.ds(s * rpt, rpt)])
        plsc.subcore_barrier()

        def run(h_ref):
            _edge_pipeline(h_ref, src_hbm, dst_hbm, acc, rows,
                           sA, dA, sB, dB, isem, gsem, s0, s1,
                           base=s * groups, pairs_k=groups // 2,
                           steps=steps, cps=cps)

        @pl.when(c == 0)
        def _():
            run(h_hbm.at[0])

        @pl.when(c == 1)
        def _():
            run(h_hbm.at[1])

        plsc.subcore_barrier()

        @pl.when(c == 0)
        def _():
            pltpu.sync_copy(acc.at[pl.ds(s * rpt, rpt)],
                            out_hbm.at[0].at[pl.ds(s * rpt, rpt)])

        @pl.when(c == 1)
        def _():
            pltpu.sync_copy(acc.at[pl.ds(s * rpt, rpt)],
                            out_hbm.at[1].at[pl.ds(s * rpt, rpt)])

    return k(h_split, src3d, dst3d, zrows)


def _sc_pool(h2v, batch3d, zpool):
    npad = h2v.shape[1]
    per_w = npad // 16
    per_w_chunks = per_w // 128

    @functools.partial(
        pl.kernel,
        out_type=jax.ShapeDtypeStruct((2, _GPAD, 32), jnp.float32),
        mesh=_mesh(),
        compiler_params=pltpu.CompilerParams(use_tc_tiling_on_sc=False),
        scratch_types=[
            pltpu.VMEM((per_w_chunks, 128), jnp.int32),
            pltpu.VMEM((per_w, 32), jnp.float32),
            pltpu.VMEM_SHARED((_GPAD, 32), jnp.float32),
        ],
    )
    def k(h_hbm, b_hbm, z_hbm, out_hbm, bidx, hrows, acc):
        c = lax.axis_index("c")
        s = lax.axis_index("s")

        @pl.when(s == 0)
        def _():
            pltpu.sync_copy(z_hbm, acc)

        plsc.subcore_barrier()

        pltpu.sync_copy(b_hbm.at[s], bidx)

        @pl.when(c == 0)
        def _():
            pltpu.sync_copy(h_hbm.at[0].at[pl.ds(s * per_w, per_w)], hrows)

        @pl.when(c == 1)
        def _():
            pltpu.sync_copy(h_hbm.at[1].at[pl.ds(s * per_w, per_w)], hrows)

        for j in range(per_w_chunks):
            pltpu.sync_copy(hrows.at[pl.ds(j * 128, 128)],
                            acc.at[bidx.at[j]], add=True)

        plsc.subcore_barrier()

        @pl.when(jnp.logical_and(c == 0, s == 0))
        def _():
            pltpu.sync_copy(acc, out_hbm.at[0])

        @pl.when(jnp.logical_and(c == 1, s == 0))
        def _():
            pltpu.sync_copy(acc, out_hbm.at[1])

    return k(h2v, batch3d, zpool)


def _pack_weights(w1, b1, w2, b2, fin):
    f32 = jnp.float32
    ii = jnp.arange(4 * fin)
    jj = jnp.arange(256)
    node_out = jj // 64
    of = jj % 64
    if fin == 64:
        node_in = (ii % 128) // 32
        feat_in = (ii // 128) * 32 + (ii % 32)
    else:
        node_in = ii // fin
        feat_in = ii % fin
    w1p = w1[feat_in][:, of] * (node_in[:, None] == node_out).astype(f32)
    ii2 = jnp.arange(256)
    node_in2 = ii2 // 64
    feat_in2 = ii2 % 64
    w2p = w2[feat_in2][:, of] * (node_in2[:, None] == node_out).astype(f32)
    return w1p, jnp.tile(b1, 4).reshape(1, 256), w2p, \
        jnp.tile(b2, 4).reshape(1, 256)


def _split_halves(z, blkp):
    lo = jnp.concatenate([z[:, k * 64:k * 64 + 32] for k in range(4)], axis=1)
    hi = jnp.concatenate([z[:, k * 64 + 32:(k + 1) * 64] for k in range(4)],
                         axis=1)
    return lo, hi


def _tc_mlp0(x_p4, a0_p4, w1p, b1p, w2p, b2p, n_valid):
    p4 = x_p4.shape[0]
    blkp = _BLK // 4

    def body(x_ref, a_ref, w1_ref, b1_ref, w2_ref, b2_ref, o_ref):
        i = pl.program_id(0)
        h = x_ref[...] + a_ref[0] + a_ref[1]
        z = jnp.dot(h, w1_ref[...], preferred_element_type=jnp.float32)
        z = jnp.maximum(z + b1_ref[...], 0.0)
        z = jnp.dot(z, w2_ref[...], preferred_element_type=jnp.float32)
        z = jnp.maximum(z + b2_ref[...], 0.0)
        node = 4 * (i * blkp + lax.broadcasted_iota(jnp.int32, (blkp, 256), 0)) \
            + lax.broadcasted_iota(jnp.int32, (blkp, 256), 1) // 64
        z = jnp.where(node < n_valid, z, 0.0)
        o_ref[0], o_ref[1] = _split_halves(z, blkp)

    return pl.pallas_call(
        body,
        grid=(p4 // blkp,),
        in_specs=[
            pl.BlockSpec((blkp, 32), lambda i: (i, 0)),
            pl.BlockSpec((2, blkp, 32), lambda i: (0, i, 0)),
            pl.BlockSpec((32, 256), lambda i: (0, 0)),
            pl.BlockSpec((1, 256), lambda i: (0, 0)),
            pl.BlockSpec((256, 256), lambda i: (0, 0)),
            pl.BlockSpec((1, 256), lambda i: (0, 0)),
        ],
        out_specs=pl.BlockSpec((2, blkp, 128), lambda i: (0, i, 0)),
        out_shape=jax.ShapeDtypeStruct((2, p4, 128), jnp.float32),
    )(x_p4, a0_p4, w1p, b1p, w2p, b2p)


def _tc_mlp_packed(h_split, a_split, w1p, b1p, w2p, b2p, n_valid):
    p4 = h_split.shape[1]
    blkp = _BLK // 4

    def body(h_ref, a_ref, w1_ref, b1_ref, w2_ref, b2_ref, o_ref):
        i = pl.program_id(0)
        h = jnp.concatenate([h_ref[0] + a_ref[0], h_ref[1] + a_ref[1]], axis=1)
        z = jnp.dot(h, w1_ref[...], preferred_element_type=jnp.float32)
        z = jnp.maximum(z + b1_ref[...], 0.0)
        z = jnp.dot(z, w2_ref[...], preferred_element_type=jnp.float32)
        z = jnp.maximum(z + b2_ref[...], 0.0)
        node = 4 * (i * blkp + lax.broadcasted_iota(jnp.int32, (blkp, 256), 0)) \
            + lax.broadcasted_iota(jnp.int32, (blkp, 256), 1) // 64
        z = jnp.where(node < n_valid, z, 0.0)
        o_ref[0], o_ref[1] = _split_halves(z, blkp)

    return pl.pallas_call(
        body,
        grid=(p4 // blkp,),
        in_specs=[
            pl.BlockSpec((2, blkp, 128), lambda i: (0, i, 0)),
            pl.BlockSpec((2, blkp, 128), lambda i: (0, i, 0)),
            pl.BlockSpec((256, 256), lambda i: (0, 0)),
            pl.BlockSpec((1, 256), lambda i: (0, 0)),
            pl.BlockSpec((256, 256), lambda i: (0, 0)),
            pl.BlockSpec((1, 256), lambda i: (0, 0)),
        ],
        out_specs=pl.BlockSpec((2, blkp, 128), lambda i: (0, i, 0)),
        out_shape=jax.ShapeDtypeStruct((2, p4, 128), jnp.float32),
    )(h_split, a_split, w1p, b1p, w2p, b2p)


def _tc_proj(pooled, w, b):
    def body(p_ref, w_ref, b_ref, o_ref):
        p = jnp.concatenate([p_ref[0, :_G], p_ref[1, :_G]], axis=1)
        o_ref[...] = jnp.dot(p, w_ref[...],
                             preferred_element_type=jnp.float32) + b_ref[...]

    return pl.pallas_call(
        body,
        out_shape=jax.ShapeDtypeStruct((_G, w.shape[1]), jnp.float32),
    )(pooled, w, b)


def kernel(x, edge_index, batch,
           l0_W1, l0_b1, l0_W2, l0_b2,
           l1_W1, l1_b1, l1_W2, l1_b2,
           l2_W1, l2_b1, l2_W2, l2_b2,
           proj_W, proj_b):
    n, in_dim = x.shape
    e = edge_index.shape[1]
    f32 = jnp.float32

    npad = ((n + 4096) // 4096) * 4096
    epad = ((e + 32767) // 32768) * 32768

    x_pad = jnp.zeros((npad, 8), f32).at[:n, :in_dim].set(x)
    fill = jnp.full((epad - e,), n, jnp.int32)
    src = jnp.concatenate([edge_index[0], fill])
    dst = jnp.concatenate([edge_index[1], fill])
    src4, dst4 = src.reshape(-1, 4, 128), dst.reshape(-1, 4, 128)
    src8, dst8 = src.reshape(-1, 8, 128), dst.reshape(-1, 8, 128)
    batch3d = jnp.concatenate(
        [batch.astype(jnp.int32), jnp.full((npad - n,), _G, jnp.int32)]
    ).reshape(16, -1, 128)
    z8 = jnp.zeros((npad // 16, 8), f32)
    z32 = jnp.zeros((npad // 16, 32), f32)
    zp = jnp.zeros((_GPAD, 32), f32)
    w1pad = jnp.zeros((8, 64), f32).at[:in_dim].set(l0_W1)
    b = lambda v: v.reshape(1, -1)
    l0w = _pack_weights(w1pad, l0_b1, l0_W2, l0_b2, fin=8)
    l1w = _pack_weights(l1_W1, l1_b1, l1_W2, l1_b2, fin=64)
    l2w = _pack_weights(l2_W1, l2_b1, l2_W2, l2_b2, fin=64)

    pk = lambda t: t.reshape(2, npad // 4, 128)
    unpk = lambda t: t.reshape(2, npad, 32)
    a0 = _sc_agg8(x_pad, src4, dst4, z8)
    h1 = _tc_mlp0(x_pad.reshape(npad // 4, 32), a0.reshape(2, npad // 4, 32),
                  *l0w, n)
    a1 = pk(_sc_agg64(unpk(h1), src8, dst8, z32))
    h2 = _tc_mlp_packed(h1, a1, *l1w, n)
    a2 = pk(_sc_agg64(unpk(h2), src8, dst8, z32))
    h3 = _tc_mlp_packed(h2, a2, *l2w, n)
    p = _sc_pool(unpk(h3), batch3d, zp)
    return _tc_proj(p, proj_W, b(proj_b))
